# baseline (speedup 1.0000x reference)
import functools

import jax
import jax.numpy as jnp
from jax import lax
from jax.experimental import pallas as pl
from jax.experimental.pallas import tpu as pltpu
from jax.experimental.pallas import tpu_sc as plsc

_D_IN = 512
_D_OUT = 512
_TOP_K = 32
_B = 4
_N = 1024
_L = 16
_NV = _N // _L
_NSUB = 32


def _score_kernel(x_ref, wt_ref, a2c_ref, s_ref, h_ref):
    h = jnp.dot(x_ref[0], wt_ref[:], preferred_element_type=jnp.float32)
    h_ref[:] = h
    s_col = jnp.dot(h_ref[:], a2c_ref[:],
                    preferred_element_type=jnp.float32)
    s_ref[0] = jnp.transpose(s_col)


def _proj_kernel(comb_ref, wt_ref, out_ref):
    row = jnp.dot(comb_ref[0], wt_ref[:],
                  preferred_element_type=jnp.float32)
    out_ref[0] = jnp.broadcast_to(row, (_N, _D_OUT))


_GDN = lax.GatherDimensionNumbers(
    offset_dims=(), collapsed_slice_dims=(0,), start_index_map=(0,))


def _shuf(x, idx):
    return lax.gather(x, idx[:, None], _GDN, slice_sizes=(1,),
                      mode=lax.GatherScatterMode.PROMISE_IN_BOUNDS)


def _bfly_max(x, lane):
    for sh in (1, 2, 4, 8):
        x = jnp.maximum(x, _shuf(x, lane ^ sh))
    return x


def _bfly_min(x, lane):
    for sh in (1, 2, 4, 8):
        x = jnp.minimum(x, _shuf(x, lane ^ sh))
    return x


def _bfly_sum(x, lane):
    for sh in (1, 2, 4, 8):
        x = x + _shuf(x, lane ^ sh)
    return x


_mesh = plsc.VectorSubcoreMesh(core_axis_name="c", subcore_axis_name="s")


@functools.partial(
    pl.kernel,
    mesh=_mesh,
    out_type=jax.ShapeDtypeStruct((_NSUB * _D_IN,), jnp.float32),
    scratch_types=[
        pltpu.VMEM((_N,), jnp.float32),
        pltpu.VMEM((_TOP_K,), jnp.int32),
        pltpu.VMEM((_TOP_K, _D_IN), jnp.float32),
        pltpu.VMEM((_D_IN,), jnp.float32),
        pltpu.SemaphoreType.DMA,
    ],
)
def _sc_topk(s_hbm, x_hbm, comb_hbm, s_work, idxv, rows, acc, sem):
    wid = lax.axis_index("s") * 2 + lax.axis_index("c")
    b = wid % _B
    pltpu.sync_copy(s_hbm.at[pl.ds(b * _N, _N)], s_work)
    lane = lax.broadcasted_iota(jnp.int32, (_L,), 0)

    def ext_body(k, carry):
        vals0, vals1, idx0, idx1 = carry
        mv = s_work[pl.ds(0, _L)]
        for v in range(1, _NV):
            mv = jnp.maximum(mv, s_work[pl.ds(_L * v, _L)])
        m = _bfly_max(mv, lane)
        iv = jnp.where(s_work[pl.ds(0, _L)] == m, lane, _N)
        for v in range(1, _NV):
            iv = jnp.minimum(
                iv, jnp.where(s_work[pl.ds(_L * v, _L)] == m,
                              lane + _L * v, _N))
        i_kv = _bfly_min(iv, lane)
        vals0 = jnp.where(lane == k, m, vals0)
        vals1 = jnp.where(lane == k - _L, m, vals1)
        idx0 = jnp.where(lane == k, i_kv, idx0)
        idx1 = jnp.where(lane == k - _L, i_kv, idx1)
        i_k = i_kv[0]
        voff = (i_k // _L) * _L
        vec = s_work[pl.ds(voff, _L)]
        s_work[pl.ds(voff, _L)] = jnp.where(lane == i_k % _L,
                                            -jnp.inf, vec)
        return vals0, vals1, idx0, idx1

    zf = jnp.zeros((_L,), jnp.float32)
    zi = jnp.zeros((_L,), jnp.int32)
    vals0, vals1, idx0, idx1 = lax.fori_loop(
        0, _TOP_K, ext_body, (zf, zf, zi, zi))

    idxv[pl.ds(0, _L)] = idx0 + b * _N
    idxv[pl.ds(_L, _L)] = idx1 + b * _N

    m0 = _shuf(vals0, lane * 0)
    e0 = jnp.exp(vals0 - m0)
    e1 = jnp.exp(vals1 - m0)
    inv = 1.0 / _bfly_sum(e0 + e1, lane)
    wvec0 = e0 * inv
    wvec1 = e1 * inv

    pltpu.async_copy(x_hbm.at[idxv], rows, sem).wait()

    for j in range(_D_IN // _L):
        accv = jnp.zeros((_L,), jnp.float32)
        for kk in range(_TOP_K):
            w_k = wvec0[kk] if kk < _L else wvec1[kk - _L]
            accv = accv + w_k * rows[kk, pl.ds(_L * j, _L)]
        acc[pl.ds(_L * j, _L)] = accv
    pltpu.sync_copy(acc, comb_hbm.at[pl.ds(wid * _D_IN, _D_IN)])


def kernel(x, W, a):
    wt = W.T
    a2c = a[:, _D_OUT:].T
    s = pl.pallas_call(
        _score_kernel,
        grid=(_B,),
        in_specs=[
            pl.BlockSpec((1, _N, _D_IN), lambda b: (b, 0, 0)),
            pl.BlockSpec((_D_IN, _D_OUT), lambda b: (0, 0)),
            pl.BlockSpec((_D_OUT, 1), lambda b: (0, 0)),
        ],
        out_specs=pl.BlockSpec((1, 1, _N), lambda b: (b, 0, 0)),
        out_shape=jax.ShapeDtypeStruct((_B, 1, _N), jnp.float32),
        scratch_shapes=[pltpu.VMEM((_N, _D_OUT), jnp.float32)],
    )(x, wt, a2c)
    comb = _sc_topk(s.reshape(_B * _N), x.reshape(_B * _N, _D_IN))
    comb = comb.reshape(_NSUB, _D_IN)[:_B]
    return pl.pallas_call(
        _proj_kernel,
        grid=(_B,),
        in_specs=[
            pl.BlockSpec((1, 1, _D_IN), lambda b: (b, 0, 0)),
            pl.BlockSpec((_D_IN, _D_OUT), lambda b: (0, 0)),
        ],
        out_specs=pl.BlockSpec((1, _N, _D_OUT), lambda b: (b, 0, 0)),
        out_shape=jax.ShapeDtypeStruct((_B, _N, _D_OUT), jnp.float32),
    )(comb.reshape(_B, 1, _D_IN), wt)


# speedup vs baseline: 2.6075x; 2.6075x over previous
import jax
import jax.numpy as jnp
from jax.experimental import pallas as pl
from jax.experimental.pallas import tpu as pltpu

_D_IN = 512
_D_OUT = 512
_TOP_K = 32
_B = 4
_N = 1024


def _gat_kernel(x_ref, wt_ref, a2c_ref, out_ref, h_ref):
    for sub in range(2):
        _one_batch(sub, x_ref, wt_ref, a2c_ref, out_ref, h_ref)


def _one_batch(sub, x_ref, wt_ref, a2c_ref, out_ref, h_ref):
    h = jnp.dot(x_ref[sub], wt_ref[:], preferred_element_type=jnp.float32)
    h_ref[sub] = h
    s_col = jnp.dot(h_ref[sub], a2c_ref[:],
                    preferred_element_type=jnp.float32)
    s = jnp.transpose(s_col)

    gt = jnp.where(s_col > s, jnp.float32(1), jnp.float32(0))
    ones = jnp.full((1, _N), jnp.float32(1))
    rank = jnp.dot(ones, gt, preferred_element_type=jnp.float32)
    sel = rank < float(_TOP_K)

    iota = jax.lax.broadcasted_iota(jnp.int32, (1, _N), 1)
    excess = jnp.sum(sel.astype(jnp.int32), axis=1, keepdims=True) - _TOP_K
    t = jnp.min(jnp.where(sel, s, jnp.inf), axis=1, keepdims=True)
    sel_f = jnp.where(sel, 1.0, 0.0)

    def fix_cond(carry):
        _, ex = carry
        return ex[0, 0] > 0

    def fix_body(carry):
        cur, ex = carry
        tied = (cur > 0.5) & (s == t)
        jmax = jnp.max(jnp.where(tied, iota, -1), axis=1, keepdims=True)
        return jnp.where(iota == jmax, 0.0, cur), ex - 1

    sel_f, _ = jax.lax.while_loop(fix_cond, fix_body, (sel_f, excess))
    sel = sel_f > 0.5

    mx = jnp.max(s, axis=1, keepdims=True)
    e = jnp.where(sel, jnp.exp(s - mx), 0.0)
    w = e / jnp.sum(e, axis=1, keepdims=True)
    row = jnp.dot(w, h_ref[sub], preferred_element_type=jnp.float32)
    out_ref[sub] = jnp.broadcast_to(row, (_N, _D_OUT))


def kernel(x, W, a):
    a2c = a[:, _D_OUT:].T
    return pl.pallas_call(
        _gat_kernel,
        grid=(_B // 2,),
        in_specs=[
            pl.BlockSpec((2, _N, _D_IN), lambda b: (b, 0, 0)),
            pl.BlockSpec((_D_IN, _D_OUT), lambda b: (0, 0)),
            pl.BlockSpec((_D_OUT, 1), lambda b: (0, 0)),
        ],
        out_specs=pl.BlockSpec((2, _N, _D_OUT), lambda b: (b, 0, 0)),
        out_shape=jax.ShapeDtypeStruct((_B, _N, _D_OUT), jnp.float32),
        scratch_shapes=[pltpu.VMEM((2, _N, _D_OUT), jnp.float32)],
    )(x, W.T, a2c)


# speedup vs baseline: 2.9745x; 1.1408x over previous
import jax
import jax.numpy as jnp
from jax.experimental import pallas as pl
from jax.experimental.pallas import tpu as pltpu

_D_IN = 512
_D_OUT = 512
_TOP_K = 32
_B = 4
_N = 1024


def _gat_kernel(x_ref, wt_ref, a2c_ref, out_ref, h_ref):
    h = jnp.dot(x_ref[0], wt_ref[:], preferred_element_type=jnp.float32)
    h_ref[:] = h
    s = jax.lax.dot_general(
        jnp.transpose(a2c_ref[:]), h_ref[:], (((1,), (1,)), ((), ())),
        preferred_element_type=jnp.float32)
    s_col = jnp.transpose(s)

    gt = jnp.where(s_col > s, jnp.float32(1), jnp.float32(0))
    ones = jnp.full((1, _N), jnp.float32(1))
    rank = jnp.dot(ones, gt, preferred_element_type=jnp.float32)
    sel = rank < float(_TOP_K)

    iota = jax.lax.broadcasted_iota(jnp.int32, (1, _N), 1)
    excess = jnp.sum(sel.astype(jnp.int32), axis=1, keepdims=True) - _TOP_K
    t = jnp.min(jnp.where(sel, s, jnp.inf), axis=1, keepdims=True)
    sel_f = jnp.where(sel, 1.0, 0.0)

    def fix_cond(carry):
        _, ex = carry
        return ex[0, 0] > 0

    def fix_body(carry):
        cur, ex = carry
        tied = (cur > 0.5) & (s == t)
        jmax = jnp.max(jnp.where(tied, iota, -1), axis=1, keepdims=True)
        return jnp.where(iota == jmax, 0.0, cur), ex - 1

    sel_f, _ = jax.lax.while_loop(fix_cond, fix_body, (sel_f, excess))
    sel = sel_f > 0.5

    mx = jnp.max(s, axis=1, keepdims=True)
    e = jnp.where(sel, jnp.exp(s - mx), 0.0)
    w = e / jnp.sum(e, axis=1, keepdims=True)
    row = jnp.dot(w, h_ref[:], preferred_element_type=jnp.float32)
    out_ref[0] = jnp.broadcast_to(row, (_N, _D_OUT))


def kernel(x, W, a):
    a2c = a[:, _D_OUT:].T
    return pl.pallas_call(
        _gat_kernel,
        grid=(_B,),
        in_specs=[
            pl.BlockSpec((1, _N, _D_IN), lambda b: (b, 0, 0)),
            pl.BlockSpec((_D_IN, _D_OUT), lambda b: (0, 0)),
            pl.BlockSpec((_D_OUT, 1), lambda b: (0, 0)),
        ],
        out_specs=pl.BlockSpec((1, _N, _D_OUT), lambda b: (b, 0, 0)),
        out_shape=jax.ShapeDtypeStruct((_B, _N, _D_OUT), jnp.float32),
        scratch_shapes=[pltpu.VMEM((_N, _D_OUT), jnp.float32)],
    )(x, W.T, a2c)
